# baseline (speedup 1.0000x reference)
import functools

import jax
import jax.numpy as jnp
import numpy as np
from jax import lax
from jax.experimental import pallas as pl
from jax.experimental.pallas import tpu as pltpu
from jax.experimental.pallas import tpu_sc as plsc

N_NODES = 10000
N_EDGES = 320000
D = 128
DP = D // 2
HID = 64

NC = 2
NS = 16
NW = NC * NS
EPW = N_EDGES // NW
CHUNK = 40
NCHUNKS = EPW // CHUNK
NSUPER = NCHUNKS // 2
PROWS = CHUNK

_HI_MASK = np.int32(-65536)

_mesh = plsc.VectorSubcoreMesh(core_axis_name="c", subcore_axis_name="s")


@functools.partial(
    pl.kernel,
    mesh=_mesh,
    out_type=jax.ShapeDtypeStruct((N_EDGES // 2, D), jnp.int32),
    scratch_types=[
        pltpu.VMEM((EPW,), jnp.int32),
        pltpu.VMEM((EPW,), jnp.int32),
        pltpu.VMEM((CHUNK, D), jnp.float32),
        pltpu.VMEM((CHUNK, D), jnp.float32),
        pltpu.VMEM((CHUNK, D), jnp.float32),
        pltpu.VMEM((CHUNK, D), jnp.float32),
        pltpu.VMEM((PROWS, D), jnp.int32),
        pltpu.VMEM((PROWS, D), jnp.int32),
        pltpu.VMEM_SHARED((N_NODES, D), jnp.float32),
        pltpu.SemaphoreType.DMA,
        pltpu.SemaphoreType.DMA,
        pltpu.SemaphoreType.DMA,
        pltpu.SemaphoreType.DMA,
        pltpu.SemaphoreType.DMA,
        pltpu.SemaphoreType.DMA,
    ],
)
def _gather_absdiff(x_hbm, src_hbm, dst_hbm, out_hbm,
                    idx_s, idx_d, a0, a1, b0, b1, o0, o1, x_sh,
                    sga0, sga1, sgb0, sgb1, sst0, sst1):
    bufs_a = (a0, a1)
    bufs_b = (b0, b1)
    bufs_o = (o0, o1)
    sems_a = (sga0, sga1)
    sems_b = (sgb0, sgb1)
    sems_o = (sst0, sst1)

    sid = lax.axis_index("s")
    wid = sid * NC + lax.axis_index("c")
    base0 = wid * EPW

    @pl.when(sid == 0)
    def _():
        pltpu.sync_copy(x_hbm, x_sh)

    plsc.subcore_barrier()

    pltpu.sync_copy(src_hbm.at[pl.ds(base0, EPW)], idx_s)
    pltpu.sync_copy(dst_hbm.at[pl.ds(base0, EPW)], idx_d)

    def issue_gathers(chunk, g):
        off = chunk * CHUNK
        pltpu.async_copy(x_sh.at[idx_s.at[pl.ds(off, CHUNK)]], bufs_a[g],
                         sems_a[g])
        pltpu.async_copy(x_sh.at[idx_d.at[pl.ds(off, CHUNK)]], bufs_b[g],
                         sems_b[g])

    def wait_gathers(g):
        pltpu.make_async_copy(x_hbm.at[pl.ds(0, CHUNK)],
                              bufs_a[g], sems_a[g]).wait()
        pltpu.make_async_copy(x_hbm.at[pl.ds(0, CHUNK)],
                              bufs_b[g], sems_b[g]).wait()

    def wait_store(oi):
        pltpu.make_async_copy(bufs_o[oi], out_hbm.at[pl.ds(0, PROWS)],
                              sems_o[oi]).wait()

    def compute_chunk(g, oi, rbase):
        buf_a, buf_b, buf_o = bufs_a[g], bufs_b[g], bufs_o[oi]

        def row_body(r, c2):
            for half in range(2):
                e = r * 2 + half
                for grp in range(DP // 16):
                    s1 = pl.ds(grp * 32, 16)
                    s2 = pl.ds(grp * 32 + 16, 16)
                    d1 = jnp.abs(buf_a[e, s1] - buf_b[e, s1])
                    d2 = jnp.abs(buf_a[e, s2] - buf_b[e, s2])
                    lo_bits = lax.shift_right_logical(
                        lax.bitcast_convert_type(d1, jnp.int32), 16)
                    hi_bits = (lax.bitcast_convert_type(d2, jnp.int32)
                               & _HI_MASK)
                    buf_o[rbase + r, pl.ds(half * DP + grp * 16, 16)] = (
                        lo_bits | hi_bits)
            return c2

        lax.fori_loop(0, CHUNK // 2, row_body, 0)

    def do_super(u, oi, first):
        c0 = u * 2
        wait_gathers(0)
        if not first:
            wait_store(oi)
        compute_chunk(0, oi, 0)

        @pl.when(c0 + 2 < NCHUNKS)
        def _():
            issue_gathers(c0 + 2, 0)

        wait_gathers(1)
        compute_chunk(1, oi, CHUNK // 2)

        @pl.when(c0 + 3 < NCHUNKS)
        def _():
            issue_gathers(c0 + 3, 1)

        pbase = wid * (EPW // 2) + u * PROWS
        pltpu.async_copy(bufs_o[oi],
                         out_hbm.at[pl.ds(pl.multiple_of(pbase, 8), PROWS)],
                         sems_o[oi])

    issue_gathers(0, 0)
    issue_gathers(1, 1)

    do_super(0, 0, first=True)
    do_super(1, 1, first=True)

    def pair_body(j, carry):
        do_super(j * 2, 0, first=False)
        do_super(j * 2 + 1, 1, first=False)
        return carry

    lax.fori_loop(1, NSUPER // 2, pair_body, 0)

    do_super(NSUPER - 1, 0, first=False)
    wait_store(1)
    wait_store(0)


BLK = 6400
BLK2 = BLK // 2
NB = N_EDGES // BLK
OROWS = BLK // 128


def _mlp_body(d_ref, w1l_ref, w1h_ref, b1_ref, w2_ref, b2_ref, o_ref):
    d32 = d_ref[...]
    lo = lax.bitcast_convert_type(d32 << 16, jnp.float32)
    hi = lax.bitcast_convert_type(d32 & _HI_MASK, jnp.float32)
    w1l = w1l_ref[...]
    w1h = w1h_ref[...]
    h_e = (jnp.dot(lo[:, :DP], w1l, preferred_element_type=jnp.float32)
           + jnp.dot(hi[:, :DP], w1h, preferred_element_type=jnp.float32))
    h_o = (jnp.dot(lo[:, DP:], w1l, preferred_element_type=jnp.float32)
           + jnp.dot(hi[:, DP:], w1h, preferred_element_type=jnp.float32))
    h_e = jnp.maximum(h_e + b1_ref[...], 0.0)
    h_o = jnp.maximum(h_o + b1_ref[...], 0.0)
    l_e = jnp.dot(h_e, w2_ref[...], preferred_element_type=jnp.float32)
    l_o = jnp.dot(h_o, w2_ref[...], preferred_element_type=jnp.float32)
    tile_e = l_e.reshape(1, OROWS // 2, 128)
    tile_o = l_o.reshape(1, OROWS // 2, 128)
    tile = jnp.concatenate([tile_e, tile_o], axis=1)
    o_ref[...] = jax.nn.sigmoid(tile + b2_ref[...])


def _mlp(diff32, w1l, w1h, b1r, w2, b2r):
    return pl.pallas_call(
        _mlp_body,
        grid=(NB,),
        in_specs=[
            pl.BlockSpec((BLK2, D), lambda g: (g, 0)),
            pl.BlockSpec((HID, HID), lambda g: (0, 0)),
            pl.BlockSpec((HID, HID), lambda g: (0, 0)),
            pl.BlockSpec((1, HID), lambda g: (0, 0)),
            pl.BlockSpec((HID, 1), lambda g: (0, 0)),
            pl.BlockSpec((1, 1), lambda g: (0, 0)),
        ],
        out_specs=pl.BlockSpec((1, OROWS, 128), lambda g: (g, 0, 0)),
        out_shape=jax.ShapeDtypeStruct((NB, OROWS, 128), jnp.float32),
    )(diff32, w1l, w1h, b1r, w2, b2r)


_PERM_LO = np.array([32 * (j // 16) + (j % 16) for j in range(DP)])
_PERM_HI = _PERM_LO + 16


def kernel(x, edge_index, W1, b1, W2, b2):
    ei = edge_index
    src = ei[0]
    dst = ei[1]
    diff32 = _gather_absdiff(x, src, dst)
    vals = _mlp(diff32, W1[_PERM_LO], W1[_PERM_HI], b1.reshape(1, HID), W2,
                b2.reshape(1, 1))
    values = vals.reshape(NB, 2, BLK2).transpose(0, 2, 1).reshape(-1)
    return (ei, values)


# speedup vs baseline: 2.1576x; 2.1576x over previous
import functools

import jax
import jax.numpy as jnp
from jax import lax
from jax.experimental import pallas as pl
from jax.experimental.pallas import tpu as pltpu
from jax.experimental.pallas import tpu_sc as plsc

N_NODES = 10000
N_EDGES = 320000
D = 128
HID = 64

NC = 2
NS = 16
NW = NC * NS
NSLICE = 5
ESL = N_EDGES // NSLICE
EPW = ESL // NW
CHUNK = 40
NCHUNKS = EPW // CHUNK
NPAIR = NCHUNKS // 2

_mesh = plsc.VectorSubcoreMesh(core_axis_name="c", subcore_axis_name="s")


@functools.partial(
    pl.kernel,
    mesh=_mesh,
    out_type=jax.ShapeDtypeStruct((ESL, D), jnp.float32),
    scratch_types=[
        pltpu.VMEM((EPW,), jnp.int32),
        pltpu.VMEM((EPW,), jnp.int32),
        pltpu.VMEM((CHUNK, D), jnp.float32),
        pltpu.VMEM((CHUNK, D), jnp.float32),
        pltpu.VMEM((CHUNK, D), jnp.float32),
        pltpu.VMEM((CHUNK, D), jnp.float32),
        pltpu.VMEM((CHUNK, D), jnp.float32),
        pltpu.VMEM((CHUNK, D), jnp.float32),
        pltpu.VMEM_SHARED((N_NODES, D), jnp.float32),
        pltpu.SemaphoreType.DMA,
        pltpu.SemaphoreType.DMA,
        pltpu.SemaphoreType.DMA,
        pltpu.SemaphoreType.DMA,
        pltpu.SemaphoreType.DMA,
        pltpu.SemaphoreType.DMA,
    ],
)
def _gather_absdiff(x_hbm, src_hbm, dst_hbm, out_hbm,
                    idx_s, idx_d, a0, a1, b0, b1, o0, o1, x_sh,
                    sga0, sga1, sgb0, sgb1, sst0, sst1):
    bufs_a = (a0, a1)
    bufs_b = (b0, b1)
    bufs_o = (o0, o1)
    sems_a = (sga0, sga1)
    sems_b = (sgb0, sgb1)
    sems_o = (sst0, sst1)

    sid = lax.axis_index("s")
    wid = sid * NC + lax.axis_index("c")
    base0 = wid * EPW

    @pl.when(sid == 0)
    def _():
        pltpu.sync_copy(x_hbm, x_sh)

    plsc.subcore_barrier()

    pltpu.sync_copy(src_hbm.at[pl.ds(base0, EPW)], idx_s)
    pltpu.sync_copy(dst_hbm.at[pl.ds(base0, EPW)], idx_d)

    def issue_gathers(chunk, s):
        off = chunk * CHUNK
        pltpu.async_copy(x_sh.at[idx_s.at[pl.ds(off, CHUNK)]], bufs_a[s],
                         sems_a[s])
        pltpu.async_copy(x_sh.at[idx_d.at[pl.ds(off, CHUNK)]], bufs_b[s],
                         sems_b[s])

    def wait_gathers(s):
        pltpu.make_async_copy(out_hbm.at[pl.ds(0, CHUNK)], bufs_a[s],
                              sems_a[s]).wait()
        pltpu.make_async_copy(out_hbm.at[pl.ds(0, CHUNK)], bufs_b[s],
                              sems_b[s]).wait()

    def wait_store(s):
        pltpu.make_async_copy(bufs_o[s], out_hbm.at[pl.ds(0, CHUNK)],
                              sems_o[s]).wait()

    issue_gathers(0, 0)
    issue_gathers(1, 1)

    def pair_body(i, carry):
        for s in (0, 1):
            chunk = i * 2 + s
            wait_gathers(s)

            @pl.when(i > 0)
            def _():
                wait_store(s)

            buf_a, buf_b, buf_o = bufs_a[s], bufs_b[s], bufs_o[s]

            def row_body(r, c2):
                for c in range(D // 16):
                    sl = pl.ds(c * 16, 16)
                    buf_o[r, sl] = jnp.abs(buf_a[r, sl] - buf_b[r, sl])
                return c2

            lax.fori_loop(0, CHUNK, row_body, 0)

            pltpu.async_copy(buf_o, out_hbm.at[pl.ds(base0 + chunk * CHUNK,
                                                     CHUNK)], sems_o[s])

            @pl.when(chunk + 2 < NCHUNKS)
            def _():
                issue_gathers(chunk + 2, s)

        return carry

    lax.fori_loop(0, NPAIR, pair_body, 0)
    wait_store(0)
    wait_store(1)


BLK = 6400
NB = ESL // BLK
OROWS = BLK // 128


def _mlp_body(d_ref, w1_ref, b1_ref, w2_ref, b2_ref, o_ref):
    h = jnp.dot(d_ref[...], w1_ref[...], preferred_element_type=jnp.float32)
    h = jnp.maximum(h + b1_ref[...], 0.0)
    logits = jnp.dot(h, w2_ref[...], preferred_element_type=jnp.float32)
    tile = logits.reshape(1, OROWS, 128)
    o_ref[...] = jax.nn.sigmoid(tile + b2_ref[...])


def _mlp(diff, w1, b1r, w2, b2r):
    return pl.pallas_call(
        _mlp_body,
        grid=(NB,),
        in_specs=[
            pl.BlockSpec((BLK, D), lambda g: (g, 0)),
            pl.BlockSpec((D, HID), lambda g: (0, 0)),
            pl.BlockSpec((1, HID), lambda g: (0, 0)),
            pl.BlockSpec((HID, 1), lambda g: (0, 0)),
            pl.BlockSpec((1, 1), lambda g: (0, 0)),
        ],
        out_specs=pl.BlockSpec((1, OROWS, 128), lambda g: (g, 0, 0)),
        out_shape=jax.ShapeDtypeStruct((NB, OROWS, 128), jnp.float32),
    )(diff, w1, b1r, w2, b2r)


def kernel(x, edge_index, W1, b1, W2, b2):
    ei = edge_index
    src = ei[0]
    dst = ei[1]
    b1r = b1.reshape(1, HID)
    b2r = b2.reshape(1, 1)
    parts = []
    for sl in range(NSLICE):
        lo = sl * ESL
        diff = _gather_absdiff(x, src[lo:lo + ESL], dst[lo:lo + ESL])
        parts.append(_mlp(diff, W1, b1r, W2, b2r).reshape(-1))
    values = jnp.concatenate(parts)
    return (ei, values)
